# Initial kernel scaffold; baseline (speedup 1.0000x reference)
import jax
import jax.numpy as jnp
from jax.experimental import pallas as pl


def kernel(boards, table, W1, b1, W2, b2, W3, b3):
    raise NotImplementedError("write your pallas kernel here")


# speedup vs baseline: 29.3228x; 29.3228x over previous
import functools
import jax
import jax.numpy as jnp
from jax import lax
from jax.experimental import pallas as pl
from jax.experimental.pallas import tpu as pltpu

_MAX_EXP = 15
_NPOS = 16
_NVAL = 16
_CDIM = _NPOS * _NVAL


def _fused_kernel(boards_ref, table_ref, W1_ref, b1_ref, W2_ref, b2_ref,
                  W3_ref, b3_ref, out_ref, T_ref):
    tile = boards_ref.shape[0]

    @pl.when(pl.program_id(0) == 0)
    def _build_T():
        tab = table_ref[:]
        for p in range(_NPOS):
            T_ref[pl.ds(p * _NVAL, _NVAL), :] = jnp.dot(
                tab, W1_ref[pl.ds(p * 32, 32), :],
                preferred_element_type=jnp.float32)

    enc = jnp.clip(boards_ref[:], 0, _MAX_EXP)

    colid = lax.broadcasted_iota(jnp.int32, (tile, _CDIM), 1)
    sel = (lax.broadcasted_iota(jnp.int32, (_NPOS, _CDIM), 0)
           == lax.broadcasted_iota(jnp.int32, (_NPOS, _CDIM), 1) // _NVAL)
    rep = jnp.dot(enc.astype(jnp.float32), sel.astype(jnp.float32),
                  preferred_element_type=jnp.float32)
    M = jnp.where(rep == (colid % _NVAL).astype(jnp.float32), 1.0, 0.0)

    h1 = jnp.maximum(
        jnp.dot(M, T_ref[:], preferred_element_type=jnp.float32) + b1_ref[:],
        0.0)
    h2 = jnp.maximum(
        jnp.dot(h1, W2_ref[:], preferred_element_type=jnp.float32) + b2_ref[:],
        0.0)
    out_ref[:] = (jnp.dot(h2, W3_ref[:], preferred_element_type=jnp.float32)
                  + b3_ref[:])


@functools.partial(jax.jit, static_argnames=("tile",))
def _run(boards, table, W1, b1, W2, b2, W3, b3, tile=1024):
    B = boards.shape[0]
    grid = (B // tile,)
    return pl.pallas_call(
        _fused_kernel,
        grid=grid,
        in_specs=[
            pl.BlockSpec((tile, _NPOS), lambda i: (i, 0)),
            pl.BlockSpec((_NVAL, 32), lambda i: (0, 0)),
            pl.BlockSpec((512, 256), lambda i: (0, 0)),
            pl.BlockSpec((1, 256), lambda i: (0, 0)),
            pl.BlockSpec((256, 256), lambda i: (0, 0)),
            pl.BlockSpec((1, 256), lambda i: (0, 0)),
            pl.BlockSpec((256, 4), lambda i: (0, 0)),
            pl.BlockSpec((1, 4), lambda i: (0, 0)),
        ],
        out_specs=pl.BlockSpec((tile, 4), lambda i: (i, 0)),
        out_shape=jax.ShapeDtypeStruct((B, 4), jnp.float32),
        scratch_shapes=[pltpu.VMEM((_CDIM, 256), jnp.float32)],
        compiler_params=pltpu.CompilerParams(
            dimension_semantics=("arbitrary",)),
    )(boards, table, W1, b1.reshape(1, 256), W2, b2.reshape(1, 256), W3,
      b3.reshape(1, 4))


def kernel(boards, table, W1, b1, W2, b2, W3, b3):
    return _run(boards, table, W1, b1, W2, b2, W3, b3)
